# Initial kernel scaffold; baseline (speedup 1.0000x reference)
import jax
import jax.numpy as jnp
from jax.experimental import pallas as pl


def kernel(x, edge_index, W0, as0, ad0, b0, W1, as1, ad1, b1, W2, as2, ad2, b2, W3, as3, ad3, b3, fcW, fcb):
    raise NotImplementedError("write your pallas kernel here")


# speedup vs baseline: 1.0871x; 1.0871x over previous
import functools

import jax
import jax.numpy as jnp
from jax.experimental import pallas as pl
from jax.experimental.pallas import tpu as pltpu

N_NODES = 10000
DIM_EMBED = 1024
ROW_BLOCK = 2000


def _matmul_body(a_ref, b_ref, o_ref):
    o_ref[...] = jnp.dot(a_ref[...], b_ref[...], preferred_element_type=jnp.float32)


def _pallas_matmul(a, b):
    m, k = a.shape
    k2, n = b.shape
    grid = m // ROW_BLOCK
    return pl.pallas_call(
        _matmul_body,
        grid=(grid,),
        in_specs=[
            pl.BlockSpec((ROW_BLOCK, k), lambda i: (i, 0)),
            pl.BlockSpec((k, n), lambda i: (0, 0)),
        ],
        out_specs=pl.BlockSpec((ROW_BLOCK, n), lambda i: (i, 0)),
        out_shape=jax.ShapeDtypeStruct((m, n), jnp.float32),
    )(a, b)


def _gat_layer(h, src, dst, W, a_s, a_d, b):
    N = h.shape[0]
    hW = _pallas_matmul(h, W)
    alpha_src = hW @ a_s
    alpha_dst = hW @ a_d
    alpha = jax.nn.leaky_relu(alpha_src[src] + alpha_dst[dst], 0.2)
    m = jax.ops.segment_max(alpha, dst, num_segments=N)
    m = jnp.where(jnp.isfinite(m), m, 0.0)
    e = jnp.exp(alpha - m[dst])
    denom = jax.ops.segment_sum(e, dst, num_segments=N)
    coef = e / (denom[dst] + 1e-16)
    out = jax.ops.segment_sum(coef[:, None] * hW[src], dst, num_segments=N)
    return out + b


def kernel(x, edge_index, W0, as0, ad0, b0, W1, as1, ad1, b1, W2, as2, ad2, b2,
           W3, as3, ad3, b3, fcW, fcb):
    N = x.shape[0]
    loop = jnp.arange(N, dtype=edge_index.dtype)
    src = jnp.concatenate([edge_index[0], loop])
    dst = jnp.concatenate([edge_index[1], loop])
    h = x
    for (W, a_s, a_d, b) in ((W0, as0, ad0, b0), (W1, as1, ad1, b1),
                             (W2, as2, ad2, b2), (W3, as3, ad3, b3)):
        h = jax.nn.relu(_gat_layer(h, src, dst, W, a_s, a_d, b))
    fcWp = jnp.zeros((DIM_EMBED, 128), jnp.float32).at[:, :fcW.shape[1]].set(fcW)
    out = _pallas_matmul(h, fcWp)[:, :fcW.shape[1]]
    return out + fcb


# speedup vs baseline: 1.1302x; 1.0397x over previous
import functools

import jax
import jax.numpy as jnp
from jax import lax
from jax.experimental import pallas as pl
from jax.experimental.pallas import tpu as pltpu
from jax.experimental.pallas import tpu_sc as plsc

N_NODES = 10000
D = 1024
ROW_BLOCK = 2000
E2 = 320000 + N_NODES
WIN = 1024
PADN = 332800
WROWS = 64
NWIN = 160
ACC_ROWS = 72


def _mm_body(a_ref, w_ref, att_ref, h_ref, ab_ref):
    h = jnp.dot(a_ref[...], w_ref[...], preferred_element_type=jnp.float32)
    h_ref[...] = h
    ab_ref[...] = jnp.dot(h, att_ref[...], preferred_element_type=jnp.float32)


def _tc_matmul(a, w, att):
    m, k = a.shape
    n = w.shape[1]
    return pl.pallas_call(
        _mm_body,
        grid=(m // ROW_BLOCK,),
        in_specs=[
            pl.BlockSpec((ROW_BLOCK, k), lambda i: (i, 0)),
            pl.BlockSpec((k, n), lambda i: (0, 0)),
            pl.BlockSpec((n, 128), lambda i: (0, 0)),
        ],
        out_specs=[
            pl.BlockSpec((ROW_BLOCK, n), lambda i: (i, 0)),
            pl.BlockSpec((ROW_BLOCK, 128), lambda i: (i, 0)),
        ],
        out_shape=[
            jax.ShapeDtypeStruct((m, n), jnp.float32),
            jax.ShapeDtypeStruct((m, 128), jnp.float32),
        ],
    )(a, w, att)


def _fc_body(a_ref, w_ref, o_ref):
    o_ref[...] = jnp.dot(a_ref[...], w_ref[...], preferred_element_type=jnp.float32)


def _tc_fc(a, w):
    m, k = a.shape
    n = w.shape[1]
    return pl.pallas_call(
        _fc_body,
        grid=(m // ROW_BLOCK,),
        in_specs=[
            pl.BlockSpec((ROW_BLOCK, k), lambda i: (i, 0)),
            pl.BlockSpec((k, n), lambda i: (0, 0)),
        ],
        out_specs=pl.BlockSpec((ROW_BLOCK, n), lambda i: (i, 0)),
        out_shape=jax.ShapeDtypeStruct((m, n), jnp.float32),
    )(a, w)


def _spmm_body(h_hbm, ssrc_hbm, sdst_hbm, scoef_hbm, bounds_hbm, zeros_hbm,
               bias_hbm, out_hbm,
               acc, bounds_v, srcbuf, dstbuf, coefbuf, rows, biasbuf,
               srcidx, gsem):
    cid = lax.axis_index("c")
    sid = lax.axis_index("s")
    wid = sid * 2 + cid
    pltpu.sync_copy(bounds_hbm, bounds_v)
    pltpu.sync_copy(bias_hbm, biasbuf)
    lanes = lax.iota(jnp.int32, 16)

    def extract(vec_f32, lane):
        return jnp.sum(jnp.where(lanes == lane, vec_f32, 0.0))

    for k in range(NWIN // 32):
        v = 32 * k + wid
        base = v * WROWS
        pltpu.sync_copy(zeros_hbm, acc.at[pl.ds(0, WROWS)])

        m = (v // 8) * 8
        bvec = bounds_v[pl.ds(m, 16)].astype(jnp.float32)
        e0 = extract(bvec, v - m).astype(jnp.int32)
        e1 = extract(bvec, v - m + 1).astype(jnp.int32)
        e0a = (e0 // 8) * 8
        nwin = (e1 - e0a + WIN - 1) // WIN

        def win_body(w, _):
            wstart = e0a + w * WIN
            pltpu.sync_copy(ssrc_hbm.at[pl.ds(wstart, WIN)], srcbuf)
            pltpu.sync_copy(sdst_hbm.at[pl.ds(wstart, WIN)], dstbuf)
            pltpu.sync_copy(scoef_hbm.at[pl.ds(wstart, WIN)], coefbuf)
            nb = jnp.maximum((jnp.minimum(e1 - wstart, WIN) + 31) // 32, 0)

            def batch_body(j, _):
                for half in range(2):
                    sl16 = pl.ds(j * 32 + half * 16, 16)
                    srcidx[pl.ds(half * 16, 16)] = srcbuf[sl16]
                pltpu.async_copy(h_hbm.at[srcidx.at[:]], rows, gsem).wait()
                for half in range(2):
                    sl16 = pl.ds(j * 32 + half * 16, 16)
                    g16 = (wstart + j * 32 + half * 16) + lanes
                    dst_v = dstbuf[sl16]
                    off_v = dst_v - base
                    valid = jnp.logical_and(
                        jnp.logical_and(g16 >= e0, g16 < e1),
                        jnp.logical_and(off_v >= 0, off_v < WROWS))
                    off_f = jnp.where(valid, off_v, WROWS).astype(jnp.float32)
                    cvec = coefbuf[sl16]

                    def acc_row(r, _):
                        cf = extract(cvec, r)
                        off_r = extract(off_f, r).astype(jnp.int32)

                        def acc_lane(l, _):
                            sl = pl.ds(l * 16, 16)
                            acc[off_r, sl] = (acc[off_r, sl]
                                              + cf * rows[half * 16 + r, sl])
                            return 0

                        lax.fori_loop(0, D // 16, acc_lane, 0, unroll=8)
                        return 0

                    lax.fori_loop(0, 16, acc_row, 0)
                return 0

            lax.fori_loop(0, nb, batch_body, 0)
            return 0

        lax.fori_loop(0, nwin, win_body, 0)

        def out_row(r, _):
            def out_lane(l, _):
                sl = pl.ds(l * 16, 16)
                acc[r, sl] = jnp.maximum(acc[r, sl] + biasbuf[sl], 0.0)
                return 0

            lax.fori_loop(0, D // 16, out_lane, 0, unroll=8)
            return 0

        lax.fori_loop(0, WROWS, out_row, 0)
        pltpu.sync_copy(acc.at[pl.ds(0, WROWS)], out_hbm.at[pl.ds(base, WROWS)])


def _sc_spmm(h, ssrc_p, sdst_p, scoef_p, bounds16, zeros_hbm, bias):
    mesh = plsc.VectorSubcoreMesh(core_axis_name="c", subcore_axis_name="s")
    f = pl.kernel(
        _spmm_body,
        out_type=jax.ShapeDtypeStruct((NWIN * WROWS, D), jnp.float32),
        mesh=mesh,
        scratch_types=[
            pltpu.VMEM((ACC_ROWS, D), jnp.float32),
            pltpu.VMEM((176,), jnp.int32),
            pltpu.VMEM((WIN,), jnp.int32),
            pltpu.VMEM((WIN,), jnp.int32),
            pltpu.VMEM((WIN,), jnp.float32),
            pltpu.VMEM((32, D), jnp.float32),
            pltpu.VMEM((D,), jnp.float32),
            pltpu.VMEM((32,), jnp.int32),
            pltpu.SemaphoreType.DMA,
        ],
        compiler_params=pltpu.CompilerParams(needs_layout_passes=False),
    )
    return f(h, ssrc_p, sdst_p, scoef_p, bounds16, zeros_hbm, bias)


def kernel(x, edge_index, W0, as0, ad0, b0, W1, as1, ad1, b1, W2, as2, ad2, b2,
           W3, as3, ad3, b3, fcW, fcb):
    N = x.shape[0]
    idt = edge_index.dtype
    loop = jnp.arange(N, dtype=idt)
    src = jnp.concatenate([edge_index[0], loop])
    dst = jnp.concatenate([edge_index[1], loop])
    order = jnp.argsort(dst)
    ssrc = src[order].astype(jnp.int32)
    sdst = dst[order].astype(jnp.int32)
    bounds = jnp.searchsorted(sdst, jnp.arange(NWIN + 1) * WROWS).astype(jnp.int32)
    bounds16 = jnp.full((176,), E2, jnp.int32).at[:NWIN + 1].set(bounds)
    ssrc_p = jnp.zeros((PADN,), jnp.int32).at[:E2].set(ssrc)
    sdst_p = jnp.full((PADN,), 1 << 20, jnp.int32).at[:E2].set(sdst)
    zeros_hbm = jnp.zeros((WROWS, D), jnp.float32)

    h = x
    for (W, a_s, a_d, b) in ((W0, as0, ad0, b0), (W1, as1, ad1, b1),
                             (W2, as2, ad2, b2), (W3, as3, ad3, b3)):
        att = jnp.zeros((W.shape[1], 128), jnp.float32)
        att = att.at[:, 0].set(a_s).at[:, 1].set(a_d)
        hW, ab = _tc_matmul(h, W, att)
        as_n = ab[:, 0]
        ad_n = ab[:, 1]
        alpha = jax.nn.leaky_relu(as_n[ssrc] + ad_n[sdst], 0.2)
        m = jax.ops.segment_max(alpha, sdst, num_segments=N)
        m = jnp.where(jnp.isfinite(m), m, 0.0)
        e = jnp.exp(alpha - m[sdst])
        denom = jax.ops.segment_sum(e, sdst, num_segments=N)
        coef = e / (denom[sdst] + 1e-16)
        scoef_p = jnp.zeros((PADN,), jnp.float32).at[:E2].set(coef)
        out = _sc_spmm(hW, ssrc_p, sdst_p, scoef_p, bounds16, zeros_hbm, b)
        h = out[:N]

    fcWp = jnp.zeros((D, 128), jnp.float32).at[:, :fcW.shape[1]].set(fcW)
    out = _tc_fc(h, fcWp)[:, :fcW.shape[1]]
    return out + fcb


# speedup vs baseline: 2.6993x; 2.3884x over previous
import functools

import jax
import jax.numpy as jnp
from jax import lax
from jax.experimental import pallas as pl
from jax.experimental.pallas import tpu as pltpu
from jax.experimental.pallas import tpu_sc as plsc

N_NODES = 10000
D = 1024
ROW_BLOCK = 2000
E2 = 320000 + N_NODES
WIN = 1024
PADN = 332800
WROWS = 64
NWIN = 160
ACC_ROWS = 72
NSRC = 10048


def _mm_body(a_ref, w_ref, att_ref, h_ref, ab_ref):
    h = jnp.dot(a_ref[...], w_ref[...], preferred_element_type=jnp.float32)
    h_ref[...] = h
    ab_ref[...] = jnp.dot(h, att_ref[...], preferred_element_type=jnp.float32)


def _tc_matmul(a, w, att):
    m, k = a.shape
    n = w.shape[1]
    return pl.pallas_call(
        _mm_body,
        grid=(m // ROW_BLOCK,),
        in_specs=[
            pl.BlockSpec((ROW_BLOCK, k), lambda i: (i, 0)),
            pl.BlockSpec((k, n), lambda i: (0, 0)),
            pl.BlockSpec((n, 128), lambda i: (0, 0)),
        ],
        out_specs=[
            pl.BlockSpec((ROW_BLOCK, n), lambda i: (i, 0)),
            pl.BlockSpec((ROW_BLOCK, 128), lambda i: (i, 0)),
        ],
        out_shape=[
            jax.ShapeDtypeStruct((m, n), jnp.float32),
            jax.ShapeDtypeStruct((m, 128), jnp.float32),
        ],
    )(a, w, att)


def _fc_body(a_ref, w_ref, o_ref):
    o_ref[...] = jnp.dot(a_ref[...], w_ref[...], preferred_element_type=jnp.float32)


def _tc_fc(a, w):
    m, k = a.shape
    n = w.shape[1]
    return pl.pallas_call(
        _fc_body,
        grid=(m // ROW_BLOCK,),
        in_specs=[
            pl.BlockSpec((ROW_BLOCK, k), lambda i: (i, 0)),
            pl.BlockSpec((k, n), lambda i: (0, 0)),
        ],
        out_specs=pl.BlockSpec((ROW_BLOCK, n), lambda i: (i, 0)),
        out_shape=jax.ShapeDtypeStruct((m, n), jnp.float32),
    )(a, w)


def _spmm_body(h_hbm, ssrc_hbm, sdst_hbm, as_hbm, ad_hbm, bounds_hbm,
               zeros_hbm, bias_hbm, out_hbm,
               acc, bounds_v, srcbuf, dstbuf, rows, biasbuf,
               srcidx, asbuf, adwin, denb, gsem):
    cid = lax.axis_index("c")
    sid = lax.axis_index("s")
    wid = sid * 2 + cid
    pltpu.sync_copy(bounds_hbm, bounds_v)
    pltpu.sync_copy(bias_hbm, biasbuf)
    pltpu.sync_copy(as_hbm, asbuf)
    lanes = lax.iota(jnp.int32, 16)
    zero16 = jnp.zeros((16,), jnp.float32)
    zero16i = jnp.zeros((16,), jnp.int32)

    def extract(vec_f32, lane):
        return jnp.sum(jnp.where(lanes == lane, vec_f32, 0.0))

    def as_max(i, mv):
        return jnp.maximum(mv, asbuf[pl.ds(i * 16, 16)])

    mas_v = lax.fori_loop(0, NSRC // 16, as_max, jnp.full((16,), -1e30))
    max_as = jnp.max(mas_v)

    for k in range(NWIN // 32):
        v = 32 * k + wid
        base = v * WROWS
        pltpu.sync_copy(zeros_hbm, acc.at[pl.ds(0, WROWS)])
        pltpu.sync_copy(ad_hbm.at[pl.ds(base, 80)], adwin)

        def den_zero(r, _):
            denb[r, pl.ds(0, 16)] = zero16
            return 0

        lax.fori_loop(0, ACC_ROWS, den_zero, 0)

        max_ad = jnp.max(jnp.maximum(
            jnp.maximum(adwin[pl.ds(0, 16)], adwin[pl.ds(16, 16)]),
            jnp.maximum(adwin[pl.ds(32, 16)], adwin[pl.ds(48, 16)])))
        s_max = max_as + max_ad
        bnd = jnp.where(s_max >= 0.0, s_max, 0.2 * s_max)

        m = (v // 8) * 8
        bvec = bounds_v[pl.ds(m, 16)].astype(jnp.float32)
        e0 = extract(bvec, v - m).astype(jnp.int32)
        e1 = extract(bvec, v - m + 1).astype(jnp.int32)
        e0a = (e0 // 8) * 8
        nwin = (e1 - e0a + WIN - 1) // WIN

        def edge_coef(sl16, g16, recompute_valid=True):
            src_v = srcbuf[sl16]
            dst_v = dstbuf[sl16]
            off_v = dst_v - base
            valid = jnp.logical_and(
                jnp.logical_and(g16 >= e0, g16 < e1),
                jnp.logical_and(off_v >= 0, off_v < WROWS))
            ad_i = jnp.clip(off_v, 0, 79)
            as_v = plsc.load_gather(asbuf, [src_v])
            ad_v = plsc.load_gather(adwin, [ad_i])
            s = as_v + ad_v
            alpha = jnp.where(s >= 0.0, s, 0.2 * s)
            ev = jnp.exp(alpha - bnd)
            offd = jnp.where(valid, off_v, WROWS)
            return ev, offd, valid

        def den_win(w, _):
            wstart = e0a + w * WIN
            pltpu.sync_copy(ssrc_hbm.at[pl.ds(wstart, WIN)], srcbuf)
            pltpu.sync_copy(sdst_hbm.at[pl.ds(wstart, WIN)], dstbuf)
            nb = jnp.maximum((jnp.minimum(e1 - wstart, WIN) + 15) // 16, 0)

            def den_batch(j, _):
                sl16 = pl.ds(j * 16, 16)
                g16 = (wstart + j * 16) + lanes
                ev, offd, valid = edge_coef(sl16, g16)
                evm = jnp.where(valid, ev, 0.0)
                off_f = offd.astype(jnp.float32)

                def den_row(r, _):
                    off_r = extract(off_f, r).astype(jnp.int32)
                    e_r = extract(evm, r)
                    denb[off_r, pl.ds(0, 16)] = denb[off_r, pl.ds(0, 16)] + e_r
                    return 0

                lax.fori_loop(0, 16, den_row, 0)
                return 0

            lax.fori_loop(0, nb, den_batch, 0)
            return 0

        lax.fori_loop(0, nwin, den_win, 0)

        def win_body(w, _):
            wstart = e0a + w * WIN
            pltpu.sync_copy(ssrc_hbm.at[pl.ds(wstart, WIN)], srcbuf)
            pltpu.sync_copy(sdst_hbm.at[pl.ds(wstart, WIN)], dstbuf)
            nb = jnp.maximum((jnp.minimum(e1 - wstart, WIN) + 31) // 32, 0)

            def batch_body(j, _):
                for half in range(2):
                    sl16 = pl.ds(j * 32 + half * 16, 16)
                    srcidx[pl.ds(half * 16, 16)] = srcbuf[sl16]
                pltpu.async_copy(h_hbm.at[srcidx.at[:]], rows, gsem).wait()
                for half in range(2):
                    sl16 = pl.ds(j * 32 + half * 16, 16)
                    g16 = (wstart + j * 32 + half * 16) + lanes
                    ev, offd, valid = edge_coef(sl16, g16)
                    den_v = plsc.load_gather(denb, [offd, zero16i])
                    cvec = ev / (den_v + 1e-16)
                    off_f = offd.astype(jnp.float32)

                    def acc_row(r, _):
                        cf = extract(cvec, r)
                        off_r = extract(off_f, r).astype(jnp.int32)

                        def acc_lane(l, _):
                            sl = pl.ds(l * 16, 16)
                            acc[off_r, sl] = (acc[off_r, sl]
                                              + cf * rows[half * 16 + r, sl])
                            return 0

                        lax.fori_loop(0, D // 16, acc_lane, 0, unroll=8)
                        return 0

                    lax.fori_loop(0, 16, acc_row, 0)
                return 0

            lax.fori_loop(0, nb, batch_body, 0)
            return 0

        lax.fori_loop(0, nwin, win_body, 0)

        def out_row(r, _):
            def out_lane(l, _):
                sl = pl.ds(l * 16, 16)
                acc[r, sl] = jnp.maximum(acc[r, sl] + biasbuf[sl], 0.0)
                return 0

            lax.fori_loop(0, D // 16, out_lane, 0, unroll=8)
            return 0

        lax.fori_loop(0, WROWS, out_row, 0)
        pltpu.sync_copy(acc.at[pl.ds(0, WROWS)], out_hbm.at[pl.ds(base, WROWS)])


def _sc_spmm(h, ssrc_p, sdst_p, as_p, ad_p, bounds16, zeros_hbm, bias):
    mesh = plsc.VectorSubcoreMesh(core_axis_name="c", subcore_axis_name="s")
    f = pl.kernel(
        _spmm_body,
        out_type=jax.ShapeDtypeStruct((NWIN * WROWS, D), jnp.float32),
        mesh=mesh,
        scratch_types=[
            pltpu.VMEM((ACC_ROWS, D), jnp.float32),
            pltpu.VMEM((176,), jnp.int32),
            pltpu.VMEM((WIN,), jnp.int32),
            pltpu.VMEM((WIN,), jnp.int32),
            pltpu.VMEM((32, D), jnp.float32),
            pltpu.VMEM((D,), jnp.float32),
            pltpu.VMEM((32,), jnp.int32),
            pltpu.VMEM((NSRC,), jnp.float32),
            pltpu.VMEM((80,), jnp.float32),
            pltpu.VMEM((ACC_ROWS, 16), jnp.float32),
            pltpu.SemaphoreType.DMA,
        ],
        compiler_params=pltpu.CompilerParams(needs_layout_passes=False),
    )
    return f(h, ssrc_p, sdst_p, as_p, ad_p, bounds16, zeros_hbm, bias)


def kernel(x, edge_index, W0, as0, ad0, b0, W1, as1, ad1, b1, W2, as2, ad2, b2,
           W3, as3, ad3, b3, fcW, fcb):
    N = x.shape[0]
    idt = edge_index.dtype
    loop = jnp.arange(N, dtype=idt)
    src = jnp.concatenate([edge_index[0], loop])
    dst = jnp.concatenate([edge_index[1], loop])
    order = jnp.argsort(dst)
    ssrc = src[order].astype(jnp.int32)
    sdst = dst[order].astype(jnp.int32)
    bounds = jnp.searchsorted(sdst, jnp.arange(NWIN + 1) * WROWS).astype(jnp.int32)
    bounds16 = jnp.full((176,), E2, jnp.int32).at[:NWIN + 1].set(bounds)
    ssrc_p = jnp.zeros((PADN,), jnp.int32).at[:E2].set(ssrc)
    sdst_p = jnp.full((PADN,), 1 << 20, jnp.int32).at[:E2].set(sdst)
    zeros_hbm = jnp.zeros((WROWS, D), jnp.float32)

    h = x
    for (W, a_s, a_d, b) in ((W0, as0, ad0, b0), (W1, as1, ad1, b1),
                             (W2, as2, ad2, b2), (W3, as3, ad3, b3)):
        att = jnp.zeros((W.shape[1], 128), jnp.float32)
        att = att.at[:, 0].set(a_s).at[:, 1].set(a_d)
        hW, ab = _tc_matmul(h, W, att)
        as_p = jnp.zeros((NSRC,), jnp.float32).at[:N].set(ab[:, 0])
        ad_p = jnp.full((10496,), -1e30, jnp.float32).at[:N].set(ab[:, 1])
        out = _sc_spmm(hW, ssrc_p, sdst_p, as_p, ad_p, bounds16, zeros_hbm, b)
        h = out[:N]

    fcWp = jnp.zeros((D, 128), jnp.float32).at[:, :fcW.shape[1]].set(fcW)
    out = _tc_fc(h, fcWp)[:, :fcW.shape[1]]
    return out + fcb


# speedup vs baseline: 3.4821x; 1.2900x over previous
import functools

import jax
import jax.numpy as jnp
from jax import lax
from jax.experimental import pallas as pl
from jax.experimental.pallas import tpu as pltpu
from jax.experimental.pallas import tpu_sc as plsc

N_NODES = 10000
D = 1024
ROW_BLOCK = 2000
E2 = 320000 + N_NODES
WIN = 1024
PADN = 332800
WROWS = 64
NWIN = 160
ACC_ROWS = 72
NSRC = 10048


def _mm_body(a_ref, w_ref, att_ref, h_ref, ab_ref):
    h = jnp.dot(a_ref[...], w_ref[...], preferred_element_type=jnp.float32)
    h_ref[...] = h
    ab_ref[...] = jnp.dot(h, att_ref[...], preferred_element_type=jnp.float32)


def _tc_matmul(a, w, att):
    m, k = a.shape
    n = w.shape[1]
    return pl.pallas_call(
        _mm_body,
        grid=(m // ROW_BLOCK,),
        in_specs=[
            pl.BlockSpec((ROW_BLOCK, k), lambda i: (i, 0)),
            pl.BlockSpec((k, n), lambda i: (0, 0)),
            pl.BlockSpec((n, 128), lambda i: (0, 0)),
        ],
        out_specs=[
            pl.BlockSpec((ROW_BLOCK, n), lambda i: (i, 0)),
            pl.BlockSpec((ROW_BLOCK, 128), lambda i: (i, 0)),
        ],
        out_shape=[
            jax.ShapeDtypeStruct((m, n), jnp.float32),
            jax.ShapeDtypeStruct((m, 128), jnp.float32),
        ],
    )(a, w, att)


def _fc_body(a_ref, w_ref, o_ref):
    o_ref[...] = jnp.dot(a_ref[...], w_ref[...], preferred_element_type=jnp.float32)


def _tc_fc(a, w):
    m, k = a.shape
    n = w.shape[1]
    return pl.pallas_call(
        _fc_body,
        grid=(m // ROW_BLOCK,),
        in_specs=[
            pl.BlockSpec((ROW_BLOCK, k), lambda i: (i, 0)),
            pl.BlockSpec((k, n), lambda i: (0, 0)),
        ],
        out_specs=pl.BlockSpec((ROW_BLOCK, n), lambda i: (i, 0)),
        out_shape=jax.ShapeDtypeStruct((m, n), jnp.float32),
    )(a, w)


def _spmm_body(h_hbm, ssrc_hbm, sdst_hbm, as_hbm, ad_hbm, bounds_hbm,
               zeros_hbm, bias_hbm, out_hbm,
               acc, bounds_v, srcbuf, dstbuf, rows, rows2, biasbuf,
               srcidx, srcidx2, asbuf, adwin, denb, gsem, gsem2):
    cid = lax.axis_index("c")
    sid = lax.axis_index("s")
    wid = sid * 2 + cid
    pltpu.sync_copy(bounds_hbm, bounds_v)
    pltpu.sync_copy(bias_hbm, biasbuf)
    pltpu.sync_copy(as_hbm, asbuf)
    lanes = lax.iota(jnp.int32, 16)
    zero16 = jnp.zeros((16,), jnp.float32)
    zero16i = jnp.zeros((16,), jnp.int32)

    def extract(vec_f32, lane):
        return jnp.sum(jnp.where(lanes == lane, vec_f32, 0.0))

    def as_max(i, mv):
        return jnp.maximum(mv, asbuf[pl.ds(i * 16, 16)])

    mas_v = lax.fori_loop(0, NSRC // 16, as_max, jnp.full((16,), -1e30))
    max_as = jnp.max(mas_v)

    for k in range(NWIN // 32):
        v = 32 * k + wid
        base = v * WROWS
        pltpu.sync_copy(zeros_hbm, acc.at[pl.ds(0, WROWS)])
        pltpu.sync_copy(ad_hbm.at[pl.ds(base, 80)], adwin)

        def den_zero(r, _):
            denb[r, pl.ds(0, 16)] = zero16
            return 0

        lax.fori_loop(0, ACC_ROWS, den_zero, 0)

        max_ad = jnp.max(jnp.maximum(
            jnp.maximum(adwin[pl.ds(0, 16)], adwin[pl.ds(16, 16)]),
            jnp.maximum(adwin[pl.ds(32, 16)], adwin[pl.ds(48, 16)])))
        s_max = max_as + max_ad
        bnd = jnp.where(s_max >= 0.0, s_max, 0.2 * s_max)

        m = (v // 8) * 8
        bvec = bounds_v[pl.ds(m, 16)].astype(jnp.float32)
        e0 = extract(bvec, v - m).astype(jnp.int32)
        e1 = extract(bvec, v - m + 1).astype(jnp.int32)
        e0a = (e0 // 8) * 8
        nwin = (e1 - e0a + WIN - 1) // WIN

        def edge_coef(sl16, g16, recompute_valid=True):
            src_v = srcbuf[sl16]
            dst_v = dstbuf[sl16]
            off_v = dst_v - base
            valid = jnp.logical_and(
                jnp.logical_and(g16 >= e0, g16 < e1),
                jnp.logical_and(off_v >= 0, off_v < WROWS))
            ad_i = jnp.clip(off_v, 0, 79)
            as_v = plsc.load_gather(asbuf, [src_v])
            ad_v = plsc.load_gather(adwin, [ad_i])
            s = as_v + ad_v
            alpha = jnp.where(s >= 0.0, s, 0.2 * s)
            ev = jnp.exp(alpha - bnd)
            offd = jnp.where(valid, off_v, WROWS)
            return ev, offd, valid

        def den_win(w, _):
            wstart = e0a + w * WIN
            pltpu.sync_copy(ssrc_hbm.at[pl.ds(wstart, WIN + 64)], srcbuf)
            pltpu.sync_copy(sdst_hbm.at[pl.ds(wstart, WIN)], dstbuf)
            nb = jnp.maximum((jnp.minimum(e1 - wstart, WIN) + 15) // 16, 0)

            def den_batch(j, _):
                sl16 = pl.ds(j * 16, 16)
                g16 = (wstart + j * 16) + lanes
                ev, offd, valid = edge_coef(sl16, g16)
                evm = jnp.where(valid, ev, 0.0)
                off_f = offd.astype(jnp.float32)

                def den_row(r, _):
                    off_r = extract(off_f, r).astype(jnp.int32)
                    e_r = extract(evm, r)
                    plsc.addupdate(denb.at[off_r, pl.ds(0, 16)], zero16 + e_r)
                    return 0

                lax.fori_loop(0, 16, den_row, 0)
                return 0

            lax.fori_loop(0, nb, den_batch, 0)
            return 0

        lax.fori_loop(0, nwin, den_win, 0)

        def win_body(w, _):
            wstart = e0a + w * WIN
            pltpu.sync_copy(ssrc_hbm.at[pl.ds(wstart, WIN + 64)], srcbuf)
            pltpu.sync_copy(sdst_hbm.at[pl.ds(wstart, WIN)], dstbuf)
            nb = jnp.maximum((jnp.minimum(e1 - wstart, WIN) + 15) // 16, 0)

            def issue(j, sidx, buf, sem):
                sidx[...] = srcbuf[pl.ds(j * 16, 16)]
                return pltpu.async_copy(h_hbm.at[sidx.at[:]], buf, sem)

            def compute(j, buf):
                sl16 = pl.ds(j * 16, 16)
                g16 = (wstart + j * 16) + lanes
                ev, offd, valid = edge_coef(sl16, g16)
                den_v = plsc.load_gather(denb, [offd, zero16i])
                cvec = ev / (den_v + 1e-16)
                off_f = offd.astype(jnp.float32)

                def acc_row(r, _):
                    cf = extract(cvec, r)
                    off_r = extract(off_f, r).astype(jnp.int32)

                    def acc_lane(l, _):
                        sl = pl.ds(l * 16, 16)
                        plsc.addupdate(acc.at[off_r, sl], cf * buf[r, sl])
                        return 0

                    lax.fori_loop(0, D // 16, acc_lane, 0, unroll=8)
                    return 0

                lax.fori_loop(0, 16, acc_row, 0)

            issue(0, srcidx, rows, gsem).wait()

            def batch_pair(p, _):
                cp = issue(2 * p + 1, srcidx2, rows2, gsem2)
                compute(2 * p, rows)
                cp.wait()
                cp2 = issue(2 * p + 2, srcidx, rows, gsem)
                compute(2 * p + 1, rows2)
                cp2.wait()
                return 0

            lax.fori_loop(0, (nb + 1) // 2, batch_pair, 0)
            return 0

        lax.fori_loop(0, nwin, win_body, 0)

        def out_row(r, _):
            def out_lane(l, _):
                sl = pl.ds(l * 16, 16)
                acc[r, sl] = jnp.maximum(acc[r, sl] + biasbuf[sl], 0.0)
                return 0

            lax.fori_loop(0, D // 16, out_lane, 0, unroll=8)
            return 0

        lax.fori_loop(0, WROWS, out_row, 0)
        pltpu.sync_copy(acc.at[pl.ds(0, WROWS)], out_hbm.at[pl.ds(base, WROWS)])


def _sc_spmm(h, ssrc_p, sdst_p, as_p, ad_p, bounds16, zeros_hbm, bias):
    mesh = plsc.VectorSubcoreMesh(core_axis_name="c", subcore_axis_name="s")
    f = pl.kernel(
        _spmm_body,
        out_type=jax.ShapeDtypeStruct((NWIN * WROWS, D), jnp.float32),
        mesh=mesh,
        scratch_types=[
            pltpu.VMEM((ACC_ROWS, D), jnp.float32),
            pltpu.VMEM((176,), jnp.int32),
            pltpu.VMEM((WIN + 64,), jnp.int32),
            pltpu.VMEM((WIN,), jnp.int32),
            pltpu.VMEM((16, D), jnp.float32),
            pltpu.VMEM((16, D), jnp.float32),
            pltpu.VMEM((D,), jnp.float32),
            pltpu.VMEM((16,), jnp.int32),
            pltpu.VMEM((16,), jnp.int32),
            pltpu.VMEM((NSRC,), jnp.float32),
            pltpu.VMEM((80,), jnp.float32),
            pltpu.VMEM((ACC_ROWS, 16), jnp.float32),
            pltpu.SemaphoreType.DMA,
            pltpu.SemaphoreType.DMA,
        ],
        compiler_params=pltpu.CompilerParams(needs_layout_passes=False),
    )
    return f(h, ssrc_p, sdst_p, as_p, ad_p, bounds16, zeros_hbm, bias)


def kernel(x, edge_index, W0, as0, ad0, b0, W1, as1, ad1, b1, W2, as2, ad2, b2,
           W3, as3, ad3, b3, fcW, fcb):
    N = x.shape[0]
    idt = edge_index.dtype
    loop = jnp.arange(N, dtype=idt)
    src = jnp.concatenate([edge_index[0], loop])
    dst = jnp.concatenate([edge_index[1], loop])
    order = jnp.argsort(dst)
    ssrc = src[order].astype(jnp.int32)
    sdst = dst[order].astype(jnp.int32)
    bounds = jnp.searchsorted(sdst, jnp.arange(NWIN + 1) * WROWS).astype(jnp.int32)
    bounds16 = jnp.full((176,), E2, jnp.int32).at[:NWIN + 1].set(bounds)
    ssrc_p = jnp.zeros((PADN,), jnp.int32).at[:E2].set(ssrc)
    sdst_p = jnp.full((PADN,), 1 << 20, jnp.int32).at[:E2].set(sdst)
    zeros_hbm = jnp.zeros((WROWS, D), jnp.float32)

    h = x
    for (W, a_s, a_d, b) in ((W0, as0, ad0, b0), (W1, as1, ad1, b1),
                             (W2, as2, ad2, b2), (W3, as3, ad3, b3)):
        att = jnp.zeros((W.shape[1], 128), jnp.float32)
        att = att.at[:, 0].set(a_s).at[:, 1].set(a_d)
        hW, ab = _tc_matmul(h, W, att)
        as_p = jnp.zeros((NSRC,), jnp.float32).at[:N].set(ab[:, 0])
        ad_p = jnp.full((10496,), -1e30, jnp.float32).at[:N].set(ab[:, 1])
        out = _sc_spmm(hW, ssrc_p, sdst_p, as_p, ad_p, bounds16, zeros_hbm, b)
        h = out[:N]

    fcWp = jnp.zeros((D, 128), jnp.float32).at[:, :fcW.shape[1]].set(fcW)
    out = _tc_fc(h, fcWp)[:, :fcW.shape[1]]
    return out + fcb


# speedup vs baseline: 4.0375x; 1.1595x over previous
import functools

import jax
import jax.numpy as jnp
from jax import lax
from jax.experimental import pallas as pl
from jax.experimental.pallas import tpu as pltpu
from jax.experimental.pallas import tpu_sc as plsc

N_NODES = 10000
D = 1024
ROW_BLOCK = 2000
E2 = 320000 + N_NODES
WIN = 1024
PADN = 332800
WROWS = 64
NWIN = 160
ACC_ROWS = 72
NSRC = 10048


def _mm_body(a_ref, w_ref, att_ref, h_ref, ab_ref):
    h = jnp.dot(a_ref[...], w_ref[...], preferred_element_type=jnp.float32)
    h_ref[...] = h
    ab_ref[...] = jnp.dot(h, att_ref[...], preferred_element_type=jnp.float32)


def _tc_matmul(a, w, att):
    m, k = a.shape
    n = w.shape[1]
    return pl.pallas_call(
        _mm_body,
        grid=(m // ROW_BLOCK,),
        in_specs=[
            pl.BlockSpec((ROW_BLOCK, k), lambda i: (i, 0)),
            pl.BlockSpec((k, n), lambda i: (0, 0)),
            pl.BlockSpec((n, 128), lambda i: (0, 0)),
        ],
        out_specs=[
            pl.BlockSpec((ROW_BLOCK, n), lambda i: (i, 0)),
            pl.BlockSpec((ROW_BLOCK, 128), lambda i: (i, 0)),
        ],
        out_shape=[
            jax.ShapeDtypeStruct((m, n), jnp.float32),
            jax.ShapeDtypeStruct((m, 128), jnp.float32),
        ],
    )(a, w, att)


def _fc_body(a_ref, w_ref, o_ref):
    o_ref[...] = jnp.dot(a_ref[...], w_ref[...], preferred_element_type=jnp.float32)


def _tc_fc(a, w):
    m, k = a.shape
    n = w.shape[1]
    return pl.pallas_call(
        _fc_body,
        grid=(m // ROW_BLOCK,),
        in_specs=[
            pl.BlockSpec((ROW_BLOCK, k), lambda i: (i, 0)),
            pl.BlockSpec((k, n), lambda i: (0, 0)),
        ],
        out_specs=pl.BlockSpec((ROW_BLOCK, n), lambda i: (i, 0)),
        out_shape=jax.ShapeDtypeStruct((m, n), jnp.float32),
    )(a, w)


def _spmm_body(h_hbm, ssrc_hbm, sdst_hbm, as_hbm, ad_hbm, bounds_hbm,
               zeros_hbm, bias_hbm, out_hbm,
               acc, bounds_v, srcbuf, dstbuf, rows, rows2, biasbuf,
               srcidx, srcidx2, asbuf, adwin, denb, gsem, gsem2):
    cid = lax.axis_index("c")
    sid = lax.axis_index("s")
    wid = sid * 2 + cid
    pltpu.sync_copy(bounds_hbm, bounds_v)
    pltpu.sync_copy(bias_hbm, biasbuf)
    pltpu.sync_copy(as_hbm, asbuf)
    lanes = lax.iota(jnp.int32, 16)
    zero16 = jnp.zeros((16,), jnp.float32)
    zero16i = jnp.zeros((16,), jnp.int32)

    def extract(vec_f32, lane):
        return jnp.sum(jnp.where(lanes == lane, vec_f32, 0.0))

    _gdn = lax.GatherDimensionNumbers(
        offset_dims=(), collapsed_slice_dims=(0,), start_index_map=(0,))

    def bcast(vec, r):
        idx = (zero16i + r).reshape(16, 1)
        return lax.gather(vec, idx, _gdn, (1,),
                          mode=lax.GatherScatterMode.PROMISE_IN_BOUNDS)

    def as_max(i, mv):
        return jnp.maximum(mv, asbuf[pl.ds(i * 16, 16)])

    mas_v = lax.fori_loop(0, NSRC // 16, as_max, jnp.full((16,), -1e30))
    max_as = jnp.max(mas_v)

    for k in range(NWIN // 32):
        v = 32 * k + wid
        base = v * WROWS
        pltpu.sync_copy(zeros_hbm, acc.at[pl.ds(0, WROWS)])
        pltpu.sync_copy(ad_hbm.at[pl.ds(base, 80)], adwin)

        def den_zero(r, _):
            denb[r, pl.ds(0, 16)] = zero16
            return 0

        lax.fori_loop(0, ACC_ROWS, den_zero, 0)

        max_ad = jnp.max(jnp.maximum(
            jnp.maximum(adwin[pl.ds(0, 16)], adwin[pl.ds(16, 16)]),
            jnp.maximum(adwin[pl.ds(32, 16)], adwin[pl.ds(48, 16)])))
        s_max = max_as + max_ad
        bnd = jnp.where(s_max >= 0.0, s_max, 0.2 * s_max)

        m = (v // 8) * 8
        bvec = bounds_v[pl.ds(m, 16)].astype(jnp.float32)
        e0 = extract(bvec, v - m).astype(jnp.int32)
        e1 = extract(bvec, v - m + 1).astype(jnp.int32)
        e0a = (e0 // 8) * 8
        nwin = (e1 - e0a + WIN - 1) // WIN

        def edge_coef(sl16, g16, recompute_valid=True):
            src_v = srcbuf[sl16]
            dst_v = dstbuf[sl16]
            off_v = dst_v - base
            valid = jnp.logical_and(
                jnp.logical_and(g16 >= e0, g16 < e1),
                jnp.logical_and(off_v >= 0, off_v < WROWS))
            ad_i = jnp.clip(off_v, 0, 79)
            as_v = plsc.load_gather(asbuf, [src_v])
            ad_v = plsc.load_gather(adwin, [ad_i])
            s = as_v + ad_v
            alpha = jnp.where(s >= 0.0, s, 0.2 * s)
            ev = jnp.exp(alpha - bnd)
            offd = jnp.where(valid, off_v, WROWS)
            return ev, offd, valid

        def den_win(w, _):
            wstart = e0a + w * WIN
            pltpu.sync_copy(ssrc_hbm.at[pl.ds(wstart, WIN + 64)], srcbuf)
            pltpu.sync_copy(sdst_hbm.at[pl.ds(wstart, WIN)], dstbuf)
            nb = jnp.maximum((jnp.minimum(e1 - wstart, WIN) + 15) // 16, 0)

            def den_batch(j, _):
                sl16 = pl.ds(j * 16, 16)
                g16 = (wstart + j * 16) + lanes
                ev, offd, valid = edge_coef(sl16, g16)
                evm = jnp.where(valid, ev, 0.0)
                plsc.addupdate_scatter(denb, [offd, lanes], evm)
                return 0

            lax.fori_loop(0, nb, den_batch, 0)
            return 0

        lax.fori_loop(0, nwin, den_win, 0)

        def den_collapse(r, _):
            row = denb[r, pl.ds(0, 16)]
            denb[r, pl.ds(0, 16)] = zero16 + jnp.sum(row)
            return 0

        lax.fori_loop(0, ACC_ROWS, den_collapse, 0)

        def win_body(w, _):
            wstart = e0a + w * WIN
            pltpu.sync_copy(ssrc_hbm.at[pl.ds(wstart, WIN + 64)], srcbuf)
            pltpu.sync_copy(sdst_hbm.at[pl.ds(wstart, WIN)], dstbuf)
            nb = jnp.maximum((jnp.minimum(e1 - wstart, WIN) + 15) // 16, 0)

            def issue(j, sidx, buf, sem):
                sidx[...] = srcbuf[pl.ds(j * 16, 16)]
                return pltpu.async_copy(h_hbm.at[sidx.at[:]], buf, sem)

            def compute(j, buf):
                sl16 = pl.ds(j * 16, 16)
                g16 = (wstart + j * 16) + lanes
                ev, offd, valid = edge_coef(sl16, g16)
                den_v = plsc.load_gather(denb, [offd, zero16i])
                cvec = ev / (den_v + 1e-16)

                def acc_row(r, _):
                    cf = bcast(cvec, r)
                    off_bc = bcast(offd, r)

                    def acc_lane(l, _):
                        sl = pl.ds(l * 16, 16)
                        plsc.addupdate_scatter(acc, [off_bc, l * 16 + lanes],
                                               cf * buf[r, sl])
                        return 0

                    lax.fori_loop(0, D // 16, acc_lane, 0, unroll=8)
                    return 0

                lax.fori_loop(0, 16, acc_row, 0)

            issue(0, srcidx, rows, gsem).wait()

            def batch_pair(p, _):
                cp = issue(2 * p + 1, srcidx2, rows2, gsem2)
                compute(2 * p, rows)
                cp.wait()
                cp2 = issue(2 * p + 2, srcidx, rows, gsem)
                compute(2 * p + 1, rows2)
                cp2.wait()
                return 0

            lax.fori_loop(0, (nb + 1) // 2, batch_pair, 0)
            return 0

        lax.fori_loop(0, nwin, win_body, 0)

        def out_row(r, _):
            def out_lane(l, _):
                sl = pl.ds(l * 16, 16)
                acc[r, sl] = jnp.maximum(acc[r, sl] + biasbuf[sl], 0.0)
                return 0

            lax.fori_loop(0, D // 16, out_lane, 0, unroll=8)
            return 0

        lax.fori_loop(0, WROWS, out_row, 0)
        pltpu.sync_copy(acc.at[pl.ds(0, WROWS)], out_hbm.at[pl.ds(base, WROWS)])


def _sc_spmm(h, ssrc_p, sdst_p, as_p, ad_p, bounds16, zeros_hbm, bias):
    mesh = plsc.VectorSubcoreMesh(core_axis_name="c", subcore_axis_name="s")
    f = pl.kernel(
        _spmm_body,
        out_type=jax.ShapeDtypeStruct((NWIN * WROWS, D), jnp.float32),
        mesh=mesh,
        scratch_types=[
            pltpu.VMEM((ACC_ROWS, D), jnp.float32),
            pltpu.VMEM((176,), jnp.int32),
            pltpu.VMEM((WIN + 64,), jnp.int32),
            pltpu.VMEM((WIN,), jnp.int32),
            pltpu.VMEM((16, D), jnp.float32),
            pltpu.VMEM((16, D), jnp.float32),
            pltpu.VMEM((D,), jnp.float32),
            pltpu.VMEM((16,), jnp.int32),
            pltpu.VMEM((16,), jnp.int32),
            pltpu.VMEM((NSRC,), jnp.float32),
            pltpu.VMEM((80,), jnp.float32),
            pltpu.VMEM((ACC_ROWS, 16), jnp.float32),
            pltpu.SemaphoreType.DMA,
            pltpu.SemaphoreType.DMA,
        ],
        compiler_params=pltpu.CompilerParams(needs_layout_passes=False),
    )
    return f(h, ssrc_p, sdst_p, as_p, ad_p, bounds16, zeros_hbm, bias)


def kernel(x, edge_index, W0, as0, ad0, b0, W1, as1, ad1, b1, W2, as2, ad2, b2,
           W3, as3, ad3, b3, fcW, fcb):
    N = x.shape[0]
    idt = edge_index.dtype
    loop = jnp.arange(N, dtype=idt)
    src = jnp.concatenate([edge_index[0], loop])
    dst = jnp.concatenate([edge_index[1], loop])
    order = jnp.argsort(dst)
    ssrc = src[order].astype(jnp.int32)
    sdst = dst[order].astype(jnp.int32)
    bounds = jnp.searchsorted(sdst, jnp.arange(NWIN + 1) * WROWS).astype(jnp.int32)
    bounds16 = jnp.full((176,), E2, jnp.int32).at[:NWIN + 1].set(bounds)
    ssrc_p = jnp.zeros((PADN,), jnp.int32).at[:E2].set(ssrc)
    sdst_p = jnp.full((PADN,), 1 << 20, jnp.int32).at[:E2].set(sdst)
    zeros_hbm = jnp.zeros((WROWS, D), jnp.float32)

    h = x
    for (W, a_s, a_d, b) in ((W0, as0, ad0, b0), (W1, as1, ad1, b1),
                             (W2, as2, ad2, b2), (W3, as3, ad3, b3)):
        att = jnp.zeros((W.shape[1], 128), jnp.float32)
        att = att.at[:, 0].set(a_s).at[:, 1].set(a_d)
        hW, ab = _tc_matmul(h, W, att)
        as_p = jnp.zeros((NSRC,), jnp.float32).at[:N].set(ab[:, 0])
        ad_p = jnp.full((10496,), -1e30, jnp.float32).at[:N].set(ab[:, 1])
        out = _sc_spmm(hW, ssrc_p, sdst_p, as_p, ad_p, bounds16, zeros_hbm, b)
        h = out[:N]

    fcWp = jnp.zeros((D, 128), jnp.float32).at[:, :fcW.shape[1]].set(fcW)
    out = _tc_fc(h, fcWp)[:, :fcW.shape[1]]
    return out + fcb
